# baseline (speedup 1.0000x reference)
import functools

import jax
import jax.numpy as jnp
from jax import lax
from jax.experimental import pallas as pl
from jax.experimental.pallas import tpu as pltpu


def _ct2d_kernel(xw_ref, w_ref, b_ref, o_ref, a_ref, y0_ref, y1_ref):
    C = xw_ref.shape[2]
    OW = o_ref.shape[3]
    bh = o_ref.shape[2] // 2
    n_hb = pl.num_programs(1) - 1
    h = pl.program_id(1)
    hb = jnp.minimum(h, n_hb - 1)
    a0 = hb * bh
    slot = lax.rem(h, 2)
    prev = lax.rem(h + 1, 2)
    bias = b_ref[...].reshape(C, 1, 1)

    for g in range(bh // 4):
        v0 = y0_ref[prev, :, pl.ds(g * 4 * OW, 4 * OW)].reshape(C, 4, OW) + bias
        v1 = y1_ref[prev, :, pl.ds(g * 4 * OW, 4 * OW)].reshape(C, 4, OW) + bias
        v = jnp.stack([v0, v1], axis=2).reshape(C, 8, OW)
        o_ref[0, :, pl.ds(8 * g, 8), :] = v

    for t in range(bh + 2):
        row = xw_ref[0, a0 + t, :, :]
        for kw in range(4):
            a_ref[pl.ds(kw * C, C), pl.ds(t * OW, OW)] = row[:, kw:kw + OW]

    n_sl = bh * OW
    s0 = a_ref[:, pl.ds(0, n_sl)]
    s1 = a_ref[:, pl.ds(OW, n_sl)]
    s2 = a_ref[:, pl.ds(2 * OW, n_sl)]
    y0_ref[slot] = jnp.dot(w_ref[0], s0, preferred_element_type=jnp.float32)
    y0_ref[slot] += jnp.dot(w_ref[1], s1, preferred_element_type=jnp.float32)
    y1_ref[slot] = jnp.dot(w_ref[2], s1, preferred_element_type=jnp.float32)
    y1_ref[slot] += jnp.dot(w_ref[3], s2, preferred_element_type=jnp.float32)


@functools.partial(jax.jit, static_argnames=("block_h",))
def _forward(x_nchw, weight, bias, *, block_h=16):
    N, C, H, W = x_nchw.shape
    OH, OW = 2 * H, 2 * W
    WD = 2 * W + 3

    bh = block_h
    while H % bh:
        bh //= 2
    n_hb = H // bh

    xt = jnp.transpose(x_nchw, (0, 2, 1, 3))
    x_il = jnp.stack([xt, jnp.zeros_like(xt)], axis=-1).reshape(N, H, C, 2 * W)
    xw = jnp.pad(x_il, ((0, 0), (1, 1), (0, 0), (2, 1))).astype(jnp.bfloat16)

    wp = []
    for dy in (0, 1):
        for di in (0, 1):
            kh = 3 - dy - 2 * di
            taps = [weight[:, :, kh, 3 - kw] for kw in range(4)]
            wp.append(jnp.stack(taps, axis=0).reshape(4 * C, C).T)
    w_all = jnp.stack(wp, axis=0).astype(jnp.bfloat16)
    b2d = bias.reshape(C, 1).astype(jnp.float32)

    return pl.pallas_call(
        _ct2d_kernel,
        out_shape=jax.ShapeDtypeStruct((N, C, OH, OW), x_nchw.dtype),
        grid=(N, n_hb + 1),
        in_specs=[
            pl.BlockSpec((1, H + 2, C, WD), lambda n, h: (n, 0, 0, 0)),
            pl.BlockSpec((4, C, 4 * C), lambda n, h: (0, 0, 0)),
            pl.BlockSpec((C, 1), lambda n, h: (0, 0)),
        ],
        out_specs=pl.BlockSpec(
            (1, C, 2 * bh, OW),
            lambda n, h: (n, 0, jnp.maximum(h - 1, 0), 0)),
        scratch_shapes=[
            pltpu.VMEM((4 * C, (bh + 2) * OW), jnp.bfloat16),
            pltpu.VMEM((2, C, bh * OW), jnp.float32),
            pltpu.VMEM((2, C, bh * OW), jnp.float32),
        ],
        compiler_params=pltpu.CompilerParams(
            dimension_semantics=("parallel", "arbitrary")),
    )(xw, w_all, b2d)


def kernel(x_nchw, weight, bias):
    return _forward(x_nchw, weight, bias)


# speedup vs baseline: 1.1357x; 1.1357x over previous
import functools

import jax
import jax.numpy as jnp
from jax import lax
from jax.experimental import pallas as pl
from jax.experimental.pallas import tpu as pltpu


def _ct2d_kernel(xw_ref, w_ref, b_ref, o_ref, a_ref, y0_ref, y1_ref):
    C = xw_ref.shape[2]
    OW = o_ref.shape[3]
    bh = o_ref.shape[2] // 2
    a0 = pl.program_id(1) * bh
    bias = b_ref[...].reshape(C, 1, 1)

    def build_row(t, carry):
        row = xw_ref[0, a0 + t, :, :]
        for kw in range(4):
            a_ref[pl.ds(kw * C, C), pl.ds(t * OW, OW)] = row[:, kw:kw + OW]
        return carry

    lax.fori_loop(0, bh + 2, build_row, 0, unroll=2)

    n_sl = bh * OW
    s0 = a_ref[:, pl.ds(0, n_sl)]
    s1 = a_ref[:, pl.ds(OW, n_sl)]
    s2 = a_ref[:, pl.ds(2 * OW, n_sl)]
    y0_ref[...] = jnp.dot(w_ref[0], s0, preferred_element_type=jnp.float32)
    y0_ref[...] += jnp.dot(w_ref[1], s1, preferred_element_type=jnp.float32)
    y1_ref[...] = jnp.dot(w_ref[2], s1, preferred_element_type=jnp.float32)
    y1_ref[...] += jnp.dot(w_ref[3], s2, preferred_element_type=jnp.float32)

    def write_grp(g, carry):
        v0 = y0_ref[:, pl.ds(g * 4 * OW, 4 * OW)].reshape(C, 4, OW) + bias
        v1 = y1_ref[:, pl.ds(g * 4 * OW, 4 * OW)].reshape(C, 4, OW) + bias
        v = jnp.stack([v0, v1], axis=2).reshape(C, 8, OW)
        o_ref[0, :, pl.ds(8 * g, 8), :] = v
        return carry

    lax.fori_loop(0, bh // 4, write_grp, 0, unroll=2)


@functools.partial(jax.jit, static_argnames=("block_h",))
def _forward(x_nchw, weight, bias, *, block_h=32):
    N, C, H, W = x_nchw.shape
    OH, OW = 2 * H, 2 * W
    WD = 2 * W + 3

    bh = block_h
    while H % bh:
        bh //= 2
    n_hb = H // bh

    xt = jnp.transpose(x_nchw, (0, 2, 1, 3))
    x_il = jnp.stack([xt, jnp.zeros_like(xt)], axis=-1).reshape(N, H, C, 2 * W)
    xw = jnp.pad(x_il, ((0, 0), (1, 1), (0, 0), (2, 1))).astype(jnp.bfloat16)

    wp = []
    for dy in (0, 1):
        for di in (0, 1):
            kh = 3 - dy - 2 * di
            taps = [weight[:, :, kh, 3 - kw] for kw in range(4)]
            wp.append(jnp.stack(taps, axis=0).reshape(4 * C, C).T)
    w_all = jnp.stack(wp, axis=0).astype(jnp.bfloat16)
    b2d = bias.reshape(C, 1).astype(jnp.float32)

    return pl.pallas_call(
        _ct2d_kernel,
        out_shape=jax.ShapeDtypeStruct((N, C, OH, OW), x_nchw.dtype),
        grid=(N, n_hb),
        in_specs=[
            pl.BlockSpec((1, H + 2, C, WD), lambda n, h: (n, 0, 0, 0)),
            pl.BlockSpec((4, C, 4 * C), lambda n, h: (0, 0, 0)),
            pl.BlockSpec((C, 1), lambda n, h: (0, 0)),
        ],
        out_specs=pl.BlockSpec((1, C, 2 * bh, OW), lambda n, h: (n, 0, h, 0)),
        scratch_shapes=[
            pltpu.VMEM((4 * C, (bh + 2) * OW), jnp.bfloat16),
            pltpu.VMEM((C, bh * OW), jnp.float32),
            pltpu.VMEM((C, bh * OW), jnp.float32),
        ],
        compiler_params=pltpu.CompilerParams(
            dimension_semantics=("parallel", "parallel")),
    )(xw, w_all, b2d)


def kernel(x_nchw, weight, bias):
    return _forward(x_nchw, weight, bias)


# speedup vs baseline: 1.1502x; 1.0128x over previous
import functools

import jax
import jax.numpy as jnp
from jax import lax
from jax.experimental import pallas as pl
from jax.experimental.pallas import tpu as pltpu


def _ct2d_kernel(xw_ref, w_ref, b_ref, o_ref, a_ref, y0_ref, y1_ref):
    C = xw_ref.shape[2]
    OW = o_ref.shape[3]
    bh = o_ref.shape[2] // 2
    a0 = pl.program_id(1) * bh
    bias = b_ref[...].reshape(C, 1, 1)

    def build_row(t, carry):
        row = xw_ref[0, a0 + t, :, :]
        for kw in range(4):
            a_ref[pl.ds(kw * C, C), pl.ds(t * OW, OW)] = row[:, kw:kw + OW]
        return carry

    lax.fori_loop(0, bh + 2, build_row, 0, unroll=2)

    n_sl = bh * OW
    s0 = a_ref[:, pl.ds(0, n_sl)]
    s1 = a_ref[:, pl.ds(OW, n_sl)]
    s2 = a_ref[:, pl.ds(2 * OW, n_sl)]
    y0_ref[...] = jnp.dot(w_ref[0], s0, preferred_element_type=jnp.float32)
    y0_ref[...] += jnp.dot(w_ref[1], s1, preferred_element_type=jnp.float32)
    y1_ref[...] = jnp.dot(w_ref[2], s1, preferred_element_type=jnp.float32)
    y1_ref[...] += jnp.dot(w_ref[3], s2, preferred_element_type=jnp.float32)

    def write_grp(g, carry):
        v0 = y0_ref[:, pl.ds(g * 4 * OW, 4 * OW)].reshape(C, 4, OW) + bias
        v1 = y1_ref[:, pl.ds(g * 4 * OW, 4 * OW)].reshape(C, 4, OW) + bias
        v = jnp.stack([v0, v1], axis=2).reshape(C, 8, OW)
        o_ref[0, :, pl.ds(8 * g, 8), :] = v
        return carry

    lax.fori_loop(0, bh // 4, write_grp, 0, unroll=2)


@functools.partial(jax.jit, static_argnames=("block_h",))
def _forward(x_nchw, weight, bias, *, block_h=64):
    N, C, H, W = x_nchw.shape
    OH, OW = 2 * H, 2 * W
    WD = 2 * W + 3

    bh = block_h
    while H % bh:
        bh //= 2
    n_hb = H // bh

    xt = jnp.transpose(x_nchw, (0, 2, 1, 3))
    x_il = jnp.stack([xt, jnp.zeros_like(xt)], axis=-1).reshape(N, H, C, 2 * W)
    xw = jnp.pad(x_il, ((0, 0), (1, 1), (0, 0), (2, 1))).astype(jnp.bfloat16)

    wp = []
    for dy in (0, 1):
        for di in (0, 1):
            kh = 3 - dy - 2 * di
            taps = [weight[:, :, kh, 3 - kw] for kw in range(4)]
            wp.append(jnp.stack(taps, axis=0).reshape(4 * C, C).T)
    w_all = jnp.stack(wp, axis=0).astype(jnp.bfloat16)
    b2d = bias.reshape(C, 1).astype(jnp.float32)

    return pl.pallas_call(
        _ct2d_kernel,
        out_shape=jax.ShapeDtypeStruct((N, C, OH, OW), x_nchw.dtype),
        grid=(N, n_hb),
        in_specs=[
            pl.BlockSpec((1, H + 2, C, WD), lambda n, h: (n, 0, 0, 0)),
            pl.BlockSpec((4, C, 4 * C), lambda n, h: (0, 0, 0)),
            pl.BlockSpec((C, 1), lambda n, h: (0, 0)),
        ],
        out_specs=pl.BlockSpec((1, C, 2 * bh, OW), lambda n, h: (n, 0, h, 0)),
        scratch_shapes=[
            pltpu.VMEM((4 * C, (bh + 2) * OW), jnp.bfloat16),
            pltpu.VMEM((C, bh * OW), jnp.float32),
            pltpu.VMEM((C, bh * OW), jnp.float32),
        ],
        compiler_params=pltpu.CompilerParams(
            dimension_semantics=("parallel", "parallel")),
    )(xw, w_all, b2d)


def kernel(x_nchw, weight, bias):
    return _forward(x_nchw, weight, bias)
